# Initial kernel scaffold; baseline (speedup 1.0000x reference)
import jax
import jax.numpy as jnp
from jax.experimental import pallas as pl


def kernel(nNeigh, atom_i_idx, atom_j_idx, dist_ij, atoms_xyz, atoms_long, atom_embedding):
    raise NotImplementedError("write your pallas kernel here")


# speedup vs baseline: 1.5691x; 1.5691x over previous
import functools

import jax
import jax.numpy as jnp
from jax import lax
from jax.experimental import pallas as pl
from jax.experimental.pallas import tpu as pltpu
from jax.experimental.pallas import tpu_sc as plsc

_SC_CORES = 2
_SC_SUBCORES = 16
_NW = _SC_CORES * _SC_SUBCORES
_CHUNK = 128


def _sc_gather(table, idx3):
    chunks_per_w = idx3.shape[1]
    rows_per_w = chunks_per_w * _CHUNK
    n_rows = _NW * rows_per_w

    mesh = plsc.VectorSubcoreMesh(core_axis_name="c", subcore_axis_name="s")

    @functools.partial(
        pl.kernel,
        mesh=mesh,
        out_type=jax.ShapeDtypeStruct((n_rows, 16), jnp.float32),
        scratch_types=[
            pltpu.VMEM((chunks_per_w, _CHUNK), jnp.int32),
            pltpu.VMEM((rows_per_w, 16), jnp.float32),
            pltpu.SemaphoreType.DMA,
        ],
        compiler_params=pltpu.CompilerParams(use_tc_tiling_on_sc=False),
    )
    def k(table_hbm, idx_hbm, out_hbm, idx_v, rows_v, sem):
        wid = lax.axis_index("s") * _SC_CORES + lax.axis_index("c")
        pltpu.sync_copy(idx_hbm.at[wid], idx_v)
        copies = []
        for g in range(chunks_per_w):
            copies.append(
                pltpu.async_copy(
                    table_hbm.at[idx_v.at[g]],
                    rows_v.at[pl.ds(g * _CHUNK, _CHUNK)],
                    sem,
                )
            )
        for cp in copies:
            cp.wait()
        pltpu.sync_copy(rows_v, out_hbm.at[pl.ds(wid * rows_per_w, rows_per_w)])

    return k(table, idx3)


def _arccos(x):
    ax = jnp.abs(x)
    p = jnp.float32(-0.0012624911)
    for coef in (0.0066700901, -0.0170881256, 0.0308918810, -0.0501743046,
                 0.0889789874, -0.2145988016, 1.5707963050):
        p = p * ax + jnp.float32(coef)
    r = jnp.sqrt(jnp.maximum(jnp.float32(1.0) - ax, jnp.float32(0.0))) * p
    return jnp.where(x >= 0, r, jnp.float32(3.14159265358979) - r)


def _tc_body(delta_ref, gi_ref, gj_ref, dist_ref, emb_ref, out_ref):
    cb = gi_ref.shape[0]
    nn = dist_ref.shape[1]
    nt = emb_ref.shape[0]
    f = emb_ref.shape[1]
    eps = jnp.float32(1e-8)

    delta = delta_ref[0]
    gi = gi_ref[...]
    gj = gj_ref[...]
    dist = dist_ref[...]
    emb = emb_ref[...]

    iota_t = lax.broadcasted_iota(jnp.int32, (1, nt), 1).astype(jnp.float32)
    ohj = (gj[:, 3:4] == iota_t).astype(jnp.float32)
    ej = lax.dot_general(ohj, emb, (((1,), (0,)), ((), ())),
                         preferred_element_type=jnp.float32)
    ohi = (gi[:, 3:4] == iota_t).astype(jnp.float32)
    ei = lax.dot_general(ohi, emb, (((1,), (0,)), ((), ())),
                         preferred_element_type=jnp.float32)

    g3 = gj.reshape(cb, nn, 16)
    vec = g3[:, :, 0:3] - gi[:, 0:3][:, None, :]
    n1 = jnp.maximum(jnp.sqrt(jnp.sum(vec * vec, axis=-1)), eps)

    ejd = ej.reshape(cb, nn, f) / dist[:, :, None]
    ei_b = jnp.broadcast_to(ei[:, None, :], (cb, nn - 1, f))

    for j in range(nn):
        def dropj(x):
            if j == 0:
                return x[:, 1:]
            if j == nn - 1:
                return x[:, : nn - 1]
            return jnp.concatenate([x[:, :j], x[:, j + 1:]], axis=1)

        dot_j = jnp.sum(vec * vec[:, j:j + 1, :], axis=-1)
        cos = dot_j / (n1[:, j:j + 1] * n1)
        ang = _arccos(cos * jnp.float32(0.9999))

        c0 = jnp.broadcast_to(dist[:, j:j + 1, None], (cb, nn - 1, 1))
        c1 = dropj(dist)[:, :, None]
        c2 = dropj(ang)[:, :, None]
        c4 = jnp.broadcast_to(ejd[:, j:j + 1, :], (cb, nn - 1, f))
        c5 = dropj(ejd)
        slab = jnp.concatenate([c0, c1, c2, ei_b, c4, c5], axis=-1) + delta
        out_ref[:, j * (nn - 1):(j + 1) * (nn - 1), :] = slab


def kernel(nNeigh, atom_i_idx, atom_j_idx, dist_ij, atoms_xyz, atoms_long, atom_embedding):
    c, nn = atom_j_idx.shape
    a = atoms_xyz.shape[0]
    nt, f = atom_embedding.shape
    d_out = 3 + 3 * f

    delta = (jnp.asarray(nNeigh, jnp.float32) - jnp.float32(nn)).reshape(1)

    table = jnp.concatenate(
        [atoms_xyz, atoms_long[:, 1:2].astype(jnp.float32),
         jnp.zeros((a, 12), jnp.float32)], axis=1)
    idx_all = jnp.concatenate([atom_i_idx, atom_j_idx.reshape(-1)])
    g = _sc_gather(table, idx_all.reshape(_NW, -1, _CHUNK))

    cb = 128
    goff = c // (cb * nn)

    ang_desc = pl.pallas_call(
        _tc_body,
        grid=(c // cb,),
        in_specs=[
            pl.BlockSpec(memory_space=pltpu.SMEM),
            pl.BlockSpec((cb, 16), lambda i: (i, 0)),
            pl.BlockSpec((cb * nn, 16), lambda i: (i + goff, 0)),
            pl.BlockSpec((cb, nn), lambda i: (i, 0)),
            pl.BlockSpec((nt, f), lambda i: (0, 0)),
        ],
        out_specs=pl.BlockSpec((cb, nn * (nn - 1), d_out), lambda i: (i, 0, 0)),
        out_shape=jax.ShapeDtypeStruct((c, nn * (nn - 1), d_out), jnp.float32),
        compiler_params=pltpu.CompilerParams(
            dimension_semantics=("parallel",)),
    )(delta, g, g, dist_ij, atom_embedding)

    return (atom_i_idx.reshape(-1), ang_desc)


# speedup vs baseline: 3.1132x; 1.9840x over previous
import functools

import jax
import jax.numpy as jnp
from jax import lax
from jax.experimental import pallas as pl
from jax.experimental.pallas import tpu as pltpu
from jax.experimental.pallas import tpu_sc as plsc

_SC_CORES = 2
_SC_SUBCORES = 16
_NW = _SC_CORES * _SC_SUBCORES
_CHUNK = 128


def _sc_gather(table, idx3):
    chunks_per_w = idx3.shape[1]
    rows_per_w = chunks_per_w * _CHUNK
    n_rows = _NW * rows_per_w

    mesh = plsc.VectorSubcoreMesh(core_axis_name="c", subcore_axis_name="s")

    @functools.partial(
        pl.kernel,
        mesh=mesh,
        out_type=jax.ShapeDtypeStruct((n_rows, 16), jnp.float32),
        scratch_types=[
            pltpu.VMEM((chunks_per_w, _CHUNK), jnp.int32),
            pltpu.VMEM((rows_per_w, 16), jnp.float32),
            pltpu.SemaphoreType.DMA,
        ],
        compiler_params=pltpu.CompilerParams(use_tc_tiling_on_sc=False),
    )
    def k(table_hbm, idx_hbm, out_hbm, idx_v, rows_v, sem):
        wid = lax.axis_index("s") * _SC_CORES + lax.axis_index("c")
        pltpu.sync_copy(idx_hbm.at[wid], idx_v)
        copies = []
        for g in range(chunks_per_w):
            copies.append(
                pltpu.async_copy(
                    table_hbm.at[idx_v.at[g]],
                    rows_v.at[pl.ds(g * _CHUNK, _CHUNK)],
                    sem,
                )
            )
        for cp in copies:
            cp.wait()
        pltpu.sync_copy(rows_v, out_hbm.at[pl.ds(wid * rows_per_w, rows_per_w)])

    return k(table, idx3)


def _arccos(x, delta):
    ax = jnp.abs(x)
    p = jnp.float32(-0.0012624911)
    for coef in (0.0066700901, -0.0170881256, 0.0308918810, -0.0501743046,
                 0.0889789874, -0.2145988016, 1.5707963050):
        p = p * ax + jnp.float32(coef)
    r = jnp.sqrt(jnp.maximum(jnp.float32(1.0) - ax, jnp.float32(0.0))) * p
    return jnp.where(x >= 0, r, jnp.float32(3.14159265358979) - r) + delta


def _tc_body(delta_ref, gi_ref, gj_ref, dist_ref, emb_ref, out_ref):
    cb = gi_ref.shape[0]
    nn = dist_ref.shape[1]
    nt = emb_ref.shape[0]
    f = emb_ref.shape[1]
    eps = jnp.float32(1e-8)

    delta = delta_ref[0]
    gi = gi_ref[...]
    gj = gj_ref[...]
    dist = dist_ref[...]
    emb = emb_ref[...]

    iota_t = lax.broadcasted_iota(jnp.int32, (1, nt), 1).astype(jnp.float32)
    ohj = (gj[:, 3:4] == iota_t).astype(jnp.float32)
    ej = lax.dot_general(ohj, emb, (((1,), (0,)), ((), ())),
                         preferred_element_type=jnp.float32)
    ohi = (gi[:, 3:4] == iota_t).astype(jnp.float32)
    ei = lax.dot_general(ohi, emb, (((1,), (0,)), ((), ())),
                         preferred_element_type=jnp.float32)

    g3 = gj.reshape(cb, nn, 16)
    g3t = jnp.swapaxes(g3, 1, 2)

    vk = [g3[:, :, d:d + 1] - gi[:, None, d:d + 1] for d in range(3)]
    nk = jnp.sqrt(vk[0] * vk[0] + vk[1] * vk[1] + vk[2] * vk[2])
    rk = jnp.float32(1.0) / jnp.maximum(nk, eps)
    vj = [g3t[:, d:d + 1, :] - gi[:, None, d:d + 1] for d in range(3)]
    nj = jnp.sqrt(vj[0] * vj[0] + vj[1] * vj[1] + vj[2] * vj[2])
    rj = jnp.float32(0.9999) / jnp.maximum(nj, eps)

    dot = vk[0] * vj[0] + vk[1] * vj[1] + vk[2] * vj[2]
    ang = _arccos(dot * rk * rj, delta)

    dist_d = dist + delta
    dist_s = dist_d[:, :, None]
    ejd = ej.reshape(cb, nn, f) / dist[:, :, None] + delta
    ei_b = jnp.broadcast_to(ei[:, None, :] + delta, (cb, nn - 1, f))

    for j in range(nn):
        def dropj(x):
            if j == 0:
                return x[:, 1:]
            if j == nn - 1:
                return x[:, : nn - 1]
            return jnp.concatenate([x[:, :j], x[:, j + 1:]], axis=1)

        c0 = jnp.broadcast_to(dist_s[:, j:j + 1], (cb, nn - 1, 1))
        c1 = dropj(dist_s)
        c2 = dropj(ang[:, :, j:j + 1])
        c4 = jnp.broadcast_to(ejd[:, j:j + 1, :], (cb, nn - 1, f))
        c5 = dropj(ejd)
        slab = jnp.concatenate([c0, c1, c2, ei_b, c4, c5], axis=-1)
        out_ref[:, j * (nn - 1):(j + 1) * (nn - 1), :] = slab


def kernel(nNeigh, atom_i_idx, atom_j_idx, dist_ij, atoms_xyz, atoms_long, atom_embedding):
    c, nn = atom_j_idx.shape
    a = atoms_xyz.shape[0]
    nt, f = atom_embedding.shape
    d_out = 3 + 3 * f

    delta = (jnp.asarray(nNeigh, jnp.float32) - jnp.float32(nn)).reshape(1)

    table = jnp.concatenate(
        [atoms_xyz, atoms_long[:, 1:2].astype(jnp.float32),
         jnp.zeros((a, 12), jnp.float32)], axis=1)
    idx_all = jnp.concatenate([atom_i_idx, atom_j_idx.reshape(-1)])
    g = _sc_gather(table, idx_all.reshape(_NW, -1, _CHUNK))

    cb = 128
    goff = c // (cb * nn)

    ang_desc = pl.pallas_call(
        _tc_body,
        grid=(c // cb,),
        in_specs=[
            pl.BlockSpec(memory_space=pltpu.SMEM),
            pl.BlockSpec((cb, 16), lambda i: (i, 0)),
            pl.BlockSpec((cb * nn, 16), lambda i: (i + goff, 0)),
            pl.BlockSpec((cb, nn), lambda i: (i, 0)),
            pl.BlockSpec((nt, f), lambda i: (0, 0)),
        ],
        out_specs=pl.BlockSpec((cb, nn * (nn - 1), d_out), lambda i: (i, 0, 0)),
        out_shape=jax.ShapeDtypeStruct((c, nn * (nn - 1), d_out), jnp.float32),
        compiler_params=pltpu.CompilerParams(
            dimension_semantics=("parallel",)),
    )(delta, g, g, dist_ij, atom_embedding)

    return (atom_i_idx.reshape(-1), ang_desc)


# speedup vs baseline: 3.1443x; 1.0100x over previous
import functools

import jax
import jax.numpy as jnp
from jax import lax
from jax.experimental import pallas as pl
from jax.experimental.pallas import tpu as pltpu
from jax.experimental.pallas import tpu_sc as plsc

_SC_CORES = 2
_SC_SUBCORES = 16
_NW = _SC_CORES * _SC_SUBCORES
_CHUNK = 128


def _sc_gather(table, idx3):
    chunks_per_w = idx3.shape[1]
    rows_per_w = chunks_per_w * _CHUNK
    n_rows = _NW * rows_per_w

    mesh = plsc.VectorSubcoreMesh(core_axis_name="c", subcore_axis_name="s")

    @functools.partial(
        pl.kernel,
        mesh=mesh,
        out_type=jax.ShapeDtypeStruct((n_rows, 16), jnp.float32),
        scratch_types=[
            pltpu.VMEM((chunks_per_w, _CHUNK), jnp.int32),
            pltpu.VMEM((rows_per_w, 16), jnp.float32),
            pltpu.SemaphoreType.DMA,
        ],
        compiler_params=pltpu.CompilerParams(use_tc_tiling_on_sc=False),
    )
    def k(table_hbm, idx_hbm, out_hbm, idx_v, rows_v, sem):
        wid = lax.axis_index("s") * _SC_CORES + lax.axis_index("c")
        pltpu.sync_copy(idx_hbm.at[wid], idx_v)
        copies = []
        for g in range(chunks_per_w):
            copies.append(
                pltpu.async_copy(
                    table_hbm.at[idx_v.at[g]],
                    rows_v.at[pl.ds(g * _CHUNK, _CHUNK)],
                    sem,
                )
            )
        for cp in copies:
            cp.wait()
        pltpu.sync_copy(rows_v, out_hbm.at[pl.ds(wid * rows_per_w, rows_per_w)])

    return k(table, idx3)


def _arccos(x, delta):
    ax = jnp.abs(x)
    p = jnp.float32(-0.0012624911)
    for coef in (0.0066700901, -0.0170881256, 0.0308918810, -0.0501743046,
                 0.0889789874, -0.2145988016, 1.5707963050):
        p = p * ax + jnp.float32(coef)
    r = jnp.sqrt(jnp.maximum(jnp.float32(1.0) - ax, jnp.float32(0.0))) * p
    return jnp.where(x >= 0, r, jnp.float32(3.14159265358979) - r) + delta


def _tc_body(delta_ref, gi_ref, gj_ref, dist_ref, emb_ref, out_ref):
    cb = gi_ref.shape[0]
    nn = dist_ref.shape[1]
    nt = emb_ref.shape[0]
    f = emb_ref.shape[1]
    eps = jnp.float32(1e-8)

    delta = delta_ref[0]
    gi = gi_ref[...]
    gj = gj_ref[...]
    dist = dist_ref[...]
    emb = emb_ref[...]

    iota_t = lax.broadcasted_iota(jnp.int32, (1, nt), 1).astype(jnp.float32)
    ohj = (gj[:, 3:4] == iota_t).astype(jnp.float32)
    ej = lax.dot_general(ohj, emb, (((1,), (0,)), ((), ())),
                         preferred_element_type=jnp.float32)
    ohi = (gi[:, 3:4] == iota_t).astype(jnp.float32)
    ei = lax.dot_general(ohi, emb, (((1,), (0,)), ((), ())),
                         preferred_element_type=jnp.float32)

    g3 = gj.reshape(cb, nn, 16)
    g3t = jnp.swapaxes(g3, 1, 2)

    vk = [g3[:, :, d:d + 1] - gi[:, None, d:d + 1] for d in range(3)]
    nk = jnp.sqrt(vk[0] * vk[0] + vk[1] * vk[1] + vk[2] * vk[2])
    rk = jnp.float32(1.0) / jnp.maximum(nk, eps)
    vj = [g3t[:, d:d + 1, :] - gi[:, None, d:d + 1] for d in range(3)]
    nj = jnp.sqrt(vj[0] * vj[0] + vj[1] * vj[1] + vj[2] * vj[2])
    rj = jnp.float32(0.9999) / jnp.maximum(nj, eps)

    dot = vk[0] * vj[0] + vk[1] * vj[1] + vk[2] * vj[2]
    ang = _arccos(dot * rk * rj, delta)

    dist_d = dist + delta
    dist_s = dist_d[:, :, None]
    ejd = ej.reshape(cb, nn, f) / dist[:, :, None] + delta
    ei_b = jnp.broadcast_to(ei[:, None, :] + delta, (cb, nn - 1, f))

    for j in range(nn):
        def dropj(x):
            if j == 0:
                return x[:, 1:]
            if j == nn - 1:
                return x[:, : nn - 1]
            return jnp.concatenate([x[:, :j], x[:, j + 1:]], axis=1)

        c0 = jnp.broadcast_to(dist_s[:, j:j + 1], (cb, nn - 1, 1))
        c1 = dropj(dist_s)
        c2 = dropj(ang[:, :, j:j + 1])
        c4 = jnp.broadcast_to(ejd[:, j:j + 1, :], (cb, nn - 1, f))
        c5 = dropj(ejd)
        slab = jnp.concatenate([c0, c1, c2, ei_b, c4, c5], axis=-1)
        out_ref[:, j * (nn - 1):(j + 1) * (nn - 1), :] = slab


def kernel(nNeigh, atom_i_idx, atom_j_idx, dist_ij, atoms_xyz, atoms_long, atom_embedding):
    c, nn = atom_j_idx.shape
    a = atoms_xyz.shape[0]
    nt, f = atom_embedding.shape
    d_out = 3 + 3 * f

    delta = (jnp.asarray(nNeigh, jnp.float32) - jnp.float32(nn)).reshape(1)

    table = jnp.concatenate(
        [atoms_xyz, atoms_long[:, 1:2].astype(jnp.float32),
         jnp.zeros((a, 12), jnp.float32)], axis=1)
    idx_all = jnp.concatenate([atom_i_idx, atom_j_idx.reshape(-1)])
    g = _sc_gather(table, idx_all.reshape(_NW, -1, _CHUNK))

    cb = 64
    goff = c // (cb * nn)

    ang_desc = pl.pallas_call(
        _tc_body,
        grid=(c // cb,),
        in_specs=[
            pl.BlockSpec(memory_space=pltpu.SMEM),
            pl.BlockSpec((cb, 16), lambda i: (i, 0)),
            pl.BlockSpec((cb * nn, 16), lambda i: (i + goff, 0)),
            pl.BlockSpec((cb, nn), lambda i: (i, 0)),
            pl.BlockSpec((nt, f), lambda i: (0, 0)),
        ],
        out_specs=pl.BlockSpec((cb, nn * (nn - 1), d_out), lambda i: (i, 0, 0)),
        out_shape=jax.ShapeDtypeStruct((c, nn * (nn - 1), d_out), jnp.float32),
        compiler_params=pltpu.CompilerParams(
            dimension_semantics=("parallel",)),
    )(delta, g, g, dist_ij, atom_embedding)

    return (atom_i_idx.reshape(-1), ang_desc)


# speedup vs baseline: 16.7075x; 5.3136x over previous
import functools

import jax
import jax.numpy as jnp
from jax import lax
from jax.experimental import pallas as pl
from jax.experimental.pallas import tpu as pltpu
from jax.experimental.pallas import tpu_sc as plsc

_SC_CORES = 2
_SC_SUBCORES = 16
_NW = _SC_CORES * _SC_SUBCORES
_CHUNK = 128


def _sc_gather(table, idx3):
    chunks_per_w = idx3.shape[1]
    rows_per_w = chunks_per_w * _CHUNK
    n_rows = _NW * rows_per_w

    mesh = plsc.VectorSubcoreMesh(core_axis_name="c", subcore_axis_name="s")

    @functools.partial(
        pl.kernel,
        mesh=mesh,
        out_type=jax.ShapeDtypeStruct((n_rows, 16), jnp.float32),
        scratch_types=[
            pltpu.VMEM((chunks_per_w, _CHUNK), jnp.int32),
            pltpu.VMEM((rows_per_w, 16), jnp.float32),
            pltpu.SemaphoreType.DMA,
        ],
        compiler_params=pltpu.CompilerParams(use_tc_tiling_on_sc=False),
    )
    def k(table_hbm, idx_hbm, out_hbm, idx_v, rows_v, sem):
        wid = lax.axis_index("s") * _SC_CORES + lax.axis_index("c")
        pltpu.sync_copy(idx_hbm.at[wid], idx_v)
        copies = []
        for g in range(chunks_per_w):
            copies.append(
                pltpu.async_copy(
                    table_hbm.at[idx_v.at[g]],
                    rows_v.at[pl.ds(g * _CHUNK, _CHUNK)],
                    sem,
                )
            )
        for cp in copies:
            cp.wait()
        pltpu.sync_copy(rows_v, out_hbm.at[pl.ds(wid * rows_per_w, rows_per_w)])

    return k(table, idx3)


def _arccos(x, delta):
    ax = jnp.abs(x)
    p = jnp.float32(-0.0012624911)
    for coef in (0.0066700901, -0.0170881256, 0.0308918810, -0.0501743046,
                 0.0889789874, -0.2145988016, 1.5707963050):
        p = p * ax + jnp.float32(coef)
    r = jnp.sqrt(jnp.maximum(jnp.float32(1.0) - ax, jnp.float32(0.0))) * p
    return jnp.where(x >= 0, r, jnp.float32(3.14159265358979) - r) + delta


def _tc_body(delta_ref, gi_ref, gj3_ref, dist_ref, emb_ref, out_ref):
    cb = gi_ref.shape[0]
    nn = dist_ref.shape[1]
    nt = emb_ref.shape[0]
    f = emb_ref.shape[1]
    eps = jnp.float32(1e-8)
    delta = delta_ref[0]

    gi_t = gi_ref[...].T
    dist_t = dist_ref[...].T
    emb_t = emb_ref[...].T
    gjn_t = [gj3_ref[n].T for n in range(nn)]
    gjt = jnp.stack(gjn_t, axis=0)

    iota_t = lax.broadcasted_iota(jnp.int32, (nt, 1), 0).astype(jnp.float32)
    rdist = jnp.float32(1.0) / dist_t
    ejd_n = []
    for n in range(nn):
        oh = (iota_t == gjt[n, 3:4, :]).astype(jnp.float32)
        ejn = lax.dot_general(emb_t, oh, (((1,), (0,)), ((), ())),
                              preferred_element_type=jnp.float32)
        ejd_n.append(ejn * rdist[n:n + 1, :] + delta)
    ejd = jnp.stack(ejd_n, axis=0)
    oh_i = (iota_t == gi_t[3:4, :]).astype(jnp.float32)
    ei = lax.dot_general(emb_t, oh_i, (((1,), (0,)), ((), ())),
                         preferred_element_type=jnp.float32) + delta

    v = [gjt[:, d, :] - gi_t[d:d + 1, :] for d in range(3)]
    n1 = jnp.sqrt(v[0] * v[0] + v[1] * v[1] + v[2] * v[2])
    rr = jnp.float32(1.0) / jnp.maximum(n1, eps)
    rr9 = rr * jnp.float32(0.9999)

    def repj(x):
        return jnp.broadcast_to(x[:, None, :], (nn, nn, cb)).reshape(nn * nn, cb)

    def repk(x):
        return jnp.broadcast_to(x[None, :, :], (nn, nn, cb)).reshape(nn * nn, cb)

    dot = (repj(v[0]) * repk(v[0]) + repj(v[1]) * repk(v[1])
           + repj(v[2]) * repk(v[2]))
    ang = _arccos(dot * repj(rr9) * repk(rr), delta)

    def dropdiag(x):
        return jnp.concatenate(
            [x[(nn + 1) * r + 1:(nn + 1) * (r + 1), :] for r in range(nn - 1)],
            axis=0)

    np_ = nn * (nn - 1)
    dist_td = dist_t + delta
    out_ref[0] = dropdiag(repj(dist_td))
    out_ref[1] = dropdiag(repk(dist_td))
    out_ref[2] = dropdiag(ang)
    for f_ in range(f):
        out_ref[3 + f_] = jnp.broadcast_to(ei[f_:f_ + 1, :], (np_, cb))
        out_ref[3 + f + f_] = dropdiag(repj(ejd[:, f_, :]))
        out_ref[3 + 2 * f + f_] = dropdiag(repk(ejd[:, f_, :]))


def kernel(nNeigh, atom_i_idx, atom_j_idx, dist_ij, atoms_xyz, atoms_long, atom_embedding):
    c, nn = atom_j_idx.shape
    a = atoms_xyz.shape[0]
    nt, f = atom_embedding.shape
    d_out = 3 + 3 * f

    delta = (jnp.asarray(nNeigh, jnp.float32) - jnp.float32(nn)).reshape(1)

    table = jnp.concatenate(
        [atoms_xyz, atoms_long[:, 1:2].astype(jnp.float32),
         jnp.zeros((a, 12), jnp.float32)], axis=1)
    idx_all = jnp.concatenate([atom_i_idx, atom_j_idx.T.reshape(-1)])
    g = _sc_gather(table, idx_all.reshape(_NW, -1, _CHUNK))
    gj3 = g[c:].reshape(nn, c, 16)

    cb = 128

    out_t = pl.pallas_call(
        _tc_body,
        grid=(c // cb,),
        in_specs=[
            pl.BlockSpec(memory_space=pltpu.SMEM),
            pl.BlockSpec((cb, 16), lambda i: (i, 0)),
            pl.BlockSpec((nn, cb, 16), lambda i: (0, i, 0)),
            pl.BlockSpec((cb, nn), lambda i: (i, 0)),
            pl.BlockSpec((nt, f), lambda i: (0, 0)),
        ],
        out_specs=pl.BlockSpec((d_out, nn * (nn - 1), cb), lambda i: (0, 0, i)),
        out_shape=jax.ShapeDtypeStruct((d_out, nn * (nn - 1), c), jnp.float32),
        compiler_params=pltpu.CompilerParams(
            dimension_semantics=("parallel",)),
    )(delta, g, gj3, dist_ij, atom_embedding)

    ang_desc = jnp.transpose(out_t, (2, 1, 0))
    return (atom_i_idx.reshape(-1), ang_desc)


# speedup vs baseline: 17.1240x; 1.0249x over previous
import functools

import jax
import jax.numpy as jnp
from jax import lax
from jax.experimental import pallas as pl
from jax.experimental.pallas import tpu as pltpu
from jax.experimental.pallas import tpu_sc as plsc

_SC_CORES = 2
_SC_SUBCORES = 16
_NW = _SC_CORES * _SC_SUBCORES
_CHUNK = 128


def _sc_gather(table, idx3):
    chunks_per_w = idx3.shape[1]
    rows_per_w = chunks_per_w * _CHUNK
    n_rows = _NW * rows_per_w

    mesh = plsc.VectorSubcoreMesh(core_axis_name="c", subcore_axis_name="s")

    @functools.partial(
        pl.kernel,
        mesh=mesh,
        out_type=jax.ShapeDtypeStruct((n_rows, 16), jnp.float32),
        scratch_types=[
            pltpu.VMEM((chunks_per_w, _CHUNK), jnp.int32),
            pltpu.VMEM((rows_per_w, 16), jnp.float32),
            pltpu.SemaphoreType.DMA,
        ],
        compiler_params=pltpu.CompilerParams(use_tc_tiling_on_sc=False),
    )
    def k(table_hbm, idx_hbm, out_hbm, idx_v, rows_v, sem):
        wid = lax.axis_index("s") * _SC_CORES + lax.axis_index("c")
        pltpu.sync_copy(idx_hbm.at[wid], idx_v)
        copies = []
        for g in range(chunks_per_w):
            copies.append(
                pltpu.async_copy(
                    table_hbm.at[idx_v.at[g]],
                    rows_v.at[pl.ds(g * _CHUNK, _CHUNK)],
                    sem,
                )
            )
        for cp in copies:
            cp.wait()
        pltpu.sync_copy(rows_v, out_hbm.at[pl.ds(wid * rows_per_w, rows_per_w)])

    return k(table, idx3)


def _arccos(x, delta):
    ax = jnp.abs(x)
    p = jnp.float32(-0.0012624911)
    for coef in (0.0066700901, -0.0170881256, 0.0308918810, -0.0501743046,
                 0.0889789874, -0.2145988016, 1.5707963050):
        p = p * ax + jnp.float32(coef)
    r = jnp.sqrt(jnp.maximum(jnp.float32(1.0) - ax, jnp.float32(0.0))) * p
    return jnp.where(x >= 0, r, jnp.float32(3.14159265358979) - r) + delta


def _tc_body(delta_ref, gi_ref, gj3_ref, dist_ref, emb_ref, out_ref):
    cb = gi_ref.shape[0]
    nn = dist_ref.shape[1]
    nt = emb_ref.shape[0]
    f = emb_ref.shape[1]
    eps = jnp.float32(1e-8)
    delta = delta_ref[0]

    gi_t = gi_ref[...].T
    dist_t = dist_ref[...].T
    emb_t = emb_ref[...].T
    gjn_t = [gj3_ref[n].T for n in range(nn)]
    gjt = jnp.stack(gjn_t, axis=0)

    iota_t = lax.broadcasted_iota(jnp.int32, (nt, 1), 0).astype(jnp.float32)
    rdist = jnp.float32(1.0) / dist_t
    ejd_n = []
    for n in range(nn):
        oh = (iota_t == gjt[n, 3:4, :]).astype(jnp.float32)
        ejn = lax.dot_general(emb_t, oh, (((1,), (0,)), ((), ())),
                              preferred_element_type=jnp.float32)
        ejd_n.append(ejn * rdist[n:n + 1, :] + delta)
    ejd = jnp.stack(ejd_n, axis=0)
    oh_i = (iota_t == gi_t[3:4, :]).astype(jnp.float32)
    ei = lax.dot_general(emb_t, oh_i, (((1,), (0,)), ((), ())),
                         preferred_element_type=jnp.float32) + delta

    v = [gjt[:, d, :] - gi_t[d:d + 1, :] for d in range(3)]
    n1 = jnp.sqrt(v[0] * v[0] + v[1] * v[1] + v[2] * v[2])
    rr = jnp.float32(1.0) / jnp.maximum(n1, eps)
    rr9 = rr * jnp.float32(0.9999)

    def repj(x):
        return jnp.broadcast_to(x[:, None, :], (nn, nn, cb)).reshape(nn * nn, cb)

    def repk(x):
        return jnp.broadcast_to(x[None, :, :], (nn, nn, cb)).reshape(nn * nn, cb)

    dot = (repj(v[0]) * repk(v[0]) + repj(v[1]) * repk(v[1])
           + repj(v[2]) * repk(v[2]))
    ang = _arccos(dot * repj(rr9) * repk(rr), delta)

    def dropdiag(x):
        return jnp.concatenate(
            [x[(nn + 1) * r + 1:(nn + 1) * (r + 1), :] for r in range(nn - 1)],
            axis=0)

    np_ = nn * (nn - 1)
    dist_td = dist_t + delta
    out_ref[0] = dropdiag(repj(dist_td))
    out_ref[1] = dropdiag(repk(dist_td))
    out_ref[2] = dropdiag(ang)
    for f_ in range(f):
        out_ref[3 + f_] = jnp.broadcast_to(ei[f_:f_ + 1, :], (np_, cb))
        out_ref[3 + f + f_] = dropdiag(repj(ejd[:, f_, :]))
        out_ref[3 + 2 * f + f_] = dropdiag(repk(ejd[:, f_, :]))


def kernel(nNeigh, atom_i_idx, atom_j_idx, dist_ij, atoms_xyz, atoms_long, atom_embedding):
    c, nn = atom_j_idx.shape
    a = atoms_xyz.shape[0]
    nt, f = atom_embedding.shape
    d_out = 3 + 3 * f

    delta = (jnp.asarray(nNeigh, jnp.float32) - jnp.float32(nn)).reshape(1)

    table = jnp.concatenate(
        [atoms_xyz, atoms_long[:, 1:2].astype(jnp.float32),
         jnp.zeros((a, 12), jnp.float32)], axis=1)
    idx_all = jnp.concatenate([atom_i_idx, atom_j_idx.T.reshape(-1)])
    g = _sc_gather(table, idx_all.reshape(_NW, -1, _CHUNK))
    gj3 = g[c:].reshape(nn, c, 16)

    cb = 256

    out_t = pl.pallas_call(
        _tc_body,
        grid=(c // cb,),
        in_specs=[
            pl.BlockSpec(memory_space=pltpu.SMEM),
            pl.BlockSpec((cb, 16), lambda i: (i, 0)),
            pl.BlockSpec((nn, cb, 16), lambda i: (0, i, 0)),
            pl.BlockSpec((cb, nn), lambda i: (i, 0)),
            pl.BlockSpec((nt, f), lambda i: (0, 0)),
        ],
        out_specs=pl.BlockSpec((d_out, nn * (nn - 1), cb), lambda i: (0, 0, i)),
        out_shape=jax.ShapeDtypeStruct((d_out, nn * (nn - 1), c), jnp.float32),
        compiler_params=pltpu.CompilerParams(
            dimension_semantics=("parallel",)),
    )(delta, g, gj3, dist_ij, atom_embedding)

    ang_desc = jnp.transpose(out_t, (2, 1, 0))
    return (atom_i_idx.reshape(-1), ang_desc)


# speedup vs baseline: 19.5508x; 1.1417x over previous
import functools

import jax
import jax.numpy as jnp
from jax import lax
from jax.experimental import pallas as pl
from jax.experimental.pallas import tpu as pltpu
from jax.experimental.pallas import tpu_sc as plsc

_SC_CORES = 2
_SC_SUBCORES = 16
_NW = _SC_CORES * _SC_SUBCORES
_CHUNK = 128


def _sc_gather(table, idx3):
    chunks_per_w = idx3.shape[1]
    rows_per_w = chunks_per_w * _CHUNK
    n_rows = _NW * rows_per_w

    mesh = plsc.VectorSubcoreMesh(core_axis_name="c", subcore_axis_name="s")

    @functools.partial(
        pl.kernel,
        mesh=mesh,
        out_type=jax.ShapeDtypeStruct((n_rows, 16), jnp.float32),
        scratch_types=[
            pltpu.VMEM((chunks_per_w, _CHUNK), jnp.int32),
            pltpu.VMEM((rows_per_w, 16), jnp.float32),
            pltpu.SemaphoreType.DMA,
        ],
        compiler_params=pltpu.CompilerParams(use_tc_tiling_on_sc=False),
    )
    def k(table_hbm, idx_hbm, out_hbm, idx_v, rows_v, sem):
        wid = lax.axis_index("s") * _SC_CORES + lax.axis_index("c")
        pltpu.sync_copy(idx_hbm.at[wid], idx_v)
        copies = []
        for g in range(chunks_per_w):
            copies.append(
                pltpu.async_copy(
                    table_hbm.at[idx_v.at[g]],
                    rows_v.at[pl.ds(g * _CHUNK, _CHUNK)],
                    sem,
                )
            )
        for cp in copies:
            cp.wait()
        pltpu.sync_copy(rows_v, out_hbm.at[pl.ds(wid * rows_per_w, rows_per_w)])

    return k(table, idx3)


def _arccos(x, delta):
    ax = jnp.abs(x)
    p = jnp.float32(-0.0012624911)
    for coef in (0.0066700901, -0.0170881256, 0.0308918810, -0.0501743046,
                 0.0889789874, -0.2145988016, 1.5707963050):
        p = p * ax + jnp.float32(coef)
    r = jnp.sqrt(jnp.maximum(jnp.float32(1.0) - ax, jnp.float32(0.0))) * p
    return jnp.where(x >= 0, r, jnp.float32(3.14159265358979) - r) + delta


def _tc_body(delta_ref, g_ref, dist_ref, emb_ref, out_ref):
    nn = dist_ref.shape[1]
    cb = g_ref.shape[0] // (nn + 1)
    nt = emb_ref.shape[0]
    f = emb_ref.shape[1]
    eps = jnp.float32(1e-8)
    delta = delta_ref[0]

    gall = g_ref[...]
    gj3 = gall[cb:].reshape(nn, cb, 16)

    gi_t = gall[0:cb].T
    dist_t = dist_ref[...].T
    emb_t = emb_ref[...].T
    gjn_t = [gj3[n].T for n in range(nn)]
    gjt = jnp.stack(gjn_t, axis=0)

    iota_t = lax.broadcasted_iota(jnp.int32, (nt, 1), 0).astype(jnp.float32)
    rdist = jnp.float32(1.0) / dist_t
    ejd_n = []
    for n in range(nn):
        oh = (iota_t == gjt[n, 3:4, :]).astype(jnp.float32)
        ejn = lax.dot_general(emb_t, oh, (((1,), (0,)), ((), ())),
                              preferred_element_type=jnp.float32)
        ejd_n.append(ejn * rdist[n:n + 1, :] + delta)
    ejd = jnp.stack(ejd_n, axis=0)
    oh_i = (iota_t == gi_t[3:4, :]).astype(jnp.float32)
    ei = lax.dot_general(emb_t, oh_i, (((1,), (0,)), ((), ())),
                         preferred_element_type=jnp.float32) + delta

    v = [gjt[:, d, :] - gi_t[d:d + 1, :] for d in range(3)]
    n1 = jnp.sqrt(v[0] * v[0] + v[1] * v[1] + v[2] * v[2])
    rr = jnp.float32(1.0) / jnp.maximum(n1, eps)
    rr9 = rr * jnp.float32(0.9999)

    def repj(x):
        return jnp.broadcast_to(x[:, None, :], (nn, nn, cb)).reshape(nn * nn, cb)

    def repk(x):
        return jnp.broadcast_to(x[None, :, :], (nn, nn, cb)).reshape(nn * nn, cb)

    dot = (repj(v[0]) * repk(v[0]) + repj(v[1]) * repk(v[1])
           + repj(v[2]) * repk(v[2]))
    ang = _arccos(dot * repj(rr9) * repk(rr), delta)

    def dropdiag(x):
        return jnp.concatenate(
            [x[(nn + 1) * r + 1:(nn + 1) * (r + 1), :] for r in range(nn - 1)],
            axis=0)

    np_ = nn * (nn - 1)
    dist_td = dist_t + delta
    out_ref[0] = dropdiag(repj(dist_td))
    out_ref[1] = dropdiag(repk(dist_td))
    out_ref[2] = dropdiag(ang)
    for f_ in range(f):
        out_ref[3 + f_] = jnp.broadcast_to(ei[f_:f_ + 1, :], (np_, cb))
        out_ref[3 + f + f_] = dropdiag(repj(ejd[:, f_, :]))
        out_ref[3 + 2 * f + f_] = dropdiag(repk(ejd[:, f_, :]))


def kernel(nNeigh, atom_i_idx, atom_j_idx, dist_ij, atoms_xyz, atoms_long, atom_embedding):
    c, nn = atom_j_idx.shape
    a = atoms_xyz.shape[0]
    nt, f = atom_embedding.shape
    d_out = 3 + 3 * f

    delta = (jnp.asarray(nNeigh, jnp.float32) - jnp.float32(nn)).reshape(1)

    table = jnp.concatenate(
        [atoms_xyz, atoms_long[:, 1:2].astype(jnp.float32),
         jnp.zeros((a, 12), jnp.float32)], axis=1)
    cb = 256
    nblk = c // cb

    ai2 = atom_i_idx.reshape(nblk, 1, cb)
    aj2 = jnp.moveaxis(atom_j_idx.T.reshape(nn, nblk, cb), 1, 0)
    idx_all = jnp.concatenate([ai2, aj2], axis=1).reshape(-1)
    g = _sc_gather(table, idx_all.reshape(_NW, -1, _CHUNK))

    out_t = pl.pallas_call(
        _tc_body,
        grid=(nblk,),
        in_specs=[
            pl.BlockSpec(memory_space=pltpu.SMEM),
            pl.BlockSpec(((nn + 1) * cb, 16), lambda i: (i, 0)),
            pl.BlockSpec((cb, nn), lambda i: (i, 0)),
            pl.BlockSpec((nt, f), lambda i: (0, 0)),
        ],
        out_specs=pl.BlockSpec((d_out, nn * (nn - 1), cb), lambda i: (0, 0, i)),
        out_shape=jax.ShapeDtypeStruct((d_out, nn * (nn - 1), c), jnp.float32),
        compiler_params=pltpu.CompilerParams(
            dimension_semantics=("parallel",)),
    )(delta, g, dist_ij, atom_embedding)

    ang_desc = jnp.transpose(out_t, (2, 1, 0))
    return (atom_i_idx.reshape(-1), ang_desc)
